# Initial kernel scaffold; baseline (speedup 1.0000x reference)
import jax
import jax.numpy as jnp
from jax.experimental import pallas as pl


def kernel(x, W1, b1, W2, b2, W3, b3, W4, b4, boosted_scores):
    raise NotImplementedError("write your pallas kernel here")


# speedup vs baseline: 18.7807x; 18.7807x over previous
import jax
import jax.numpy as jnp
from jax.experimental import pallas as pl
from jax.experimental.pallas import tpu as pltpu

IN_DIM = 784
INTER = 512
SD = 16
NS = 256
HID = SD * NS
K_NEURONS = 256
K_STRIPES = 32
BETA = 1.5
GAMMA = 0

BLK = 256


def _kth_thresh(bits, k, rows):
    T = jnp.zeros((rows, 1), jnp.int32)
    for b in range(30, -1, -1):
        cand = T | jnp.int32(1 << b)
        cnt = jnp.sum((bits >= cand).astype(jnp.float32), axis=1, keepdims=True)
        T = jnp.where(cnt >= k, cand, T)
    return T


def _fused(x_ref, w1_ref, b1_ref, w2_ref, b2_ref, w3_ref, b3_ref, w4_ref,
           b4_ref, boosts_ref, s_ref, st_ref, out_ref):
    x = x_ref[...]
    h1 = jnp.maximum(
        jnp.dot(x, w1_ref[...], preferred_element_type=jnp.float32) + b1_ref[...], 0.0)
    h2 = jnp.maximum(
        jnp.dot(h1, w2_ref[...], preferred_element_type=jnp.float32) + b2_ref[...], 0.0)

    boosted = h2 * boosts_ref[...]
    bits = jax.lax.bitcast_convert_type(boosted, jnp.int32)
    T = _kth_thresh(bits, K_NEURONS, BLK)
    hm = jnp.where(bits >= T, h2, 0.0)

    ssum = jnp.dot(hm, s_ref[...], preferred_element_type=jnp.float32,
                   precision=jax.lax.Precision.HIGHEST)
    sbits = jax.lax.bitcast_convert_type(ssum, jnp.int32)
    T2 = _kth_thresh(sbits, K_STRIPES, BLK)
    smask = (sbits >= T2).astype(jnp.float32)
    sexp = jnp.dot(smask, st_ref[...], preferred_element_type=jnp.float32)
    hf = hm * sexp

    d = jnp.maximum(
        jnp.dot(hf, w3_ref[...], preferred_element_type=jnp.float32) + b3_ref[...], 0.0)
    out_ref[...] = jnp.maximum(
        jnp.dot(d, w4_ref[...], preferred_element_type=jnp.float32) + b4_ref[...], 0.0)


def kernel(x, W1, b1, W2, b2, W3, b3, W4, b4, boosted_scores):
    B = x.shape[0]
    grid = B // BLK
    boosts = jnp.exp(BETA * (GAMMA - boosted_scores)).reshape(1, HID)
    stripe_of = jnp.arange(HID, dtype=jnp.int32) // SD
    S = (stripe_of[:, None] == jnp.arange(NS, dtype=jnp.int32)[None, :]).astype(jnp.float32)

    full = lambda shape: pl.BlockSpec(shape, lambda i: (0, 0))
    out = pl.pallas_call(
        _fused,
        grid=(grid,),
        in_specs=[
            pl.BlockSpec((BLK, IN_DIM), lambda i: (i, 0)),
            full((IN_DIM, INTER)),
            full((1, INTER)),
            full((INTER, HID)),
            full((1, HID)),
            full((HID, INTER)),
            full((1, INTER)),
            full((INTER, IN_DIM)),
            full((1, IN_DIM)),
            full((1, HID)),
            full((HID, NS)),
            full((NS, HID)),
        ],
        out_specs=pl.BlockSpec((BLK, IN_DIM), lambda i: (i, 0)),
        out_shape=jax.ShapeDtypeStruct((B, IN_DIM), jnp.float32),
        compiler_params=pltpu.CompilerParams(
            dimension_semantics=("arbitrary",),
        ),
    )(x, W1.T, b1.reshape(1, INTER), W2.T, b2.reshape(1, HID),
      W3.T, b3.reshape(1, INTER), W4.T, b4.reshape(1, IN_DIM),
      boosts, S, S.T)
    return out


# speedup vs baseline: 21.6986x; 1.1554x over previous
import jax
import jax.numpy as jnp
from jax.experimental import pallas as pl
from jax.experimental.pallas import tpu as pltpu

IN_DIM = 784
INTER = 512
SD = 16
NS = 256
HID = SD * NS
K_NEURONS = 256
K_STRIPES = 32
BETA = 1.5
GAMMA = 0

BLK = 256


def _kth_thresh(bits, k, rows):
    T = jnp.zeros((rows, 1), jnp.int32)
    for b in range(30, -1, -1):
        cand = T | jnp.int32(1 << b)
        cnt = jnp.sum((bits >= cand).astype(jnp.float32), axis=1, keepdims=True)
        T = jnp.where(cnt >= k, cand, T)
    return T


def _fused(x_ref, w1_ref, b1_ref, w2_ref, b2_ref, w3_ref, b3_ref, w4_ref,
           b4_ref, boosts_ref, s_ref, st_ref, out_ref):
    x = x_ref[...]
    h1 = jnp.maximum(
        jnp.dot(x, w1_ref[...], preferred_element_type=jnp.float32) + b1_ref[...], 0.0)
    h2 = jnp.maximum(
        jnp.dot(h1, w2_ref[...], preferred_element_type=jnp.float32) + b2_ref[...], 0.0)

    boosted = h2 * boosts_ref[...]
    bits = jax.lax.bitcast_convert_type(boosted, jnp.int32)
    T = _kth_thresh(bits, K_NEURONS, BLK)
    hm = jnp.where(bits >= T, h2, 0.0)

    hm_hi = hm.astype(jnp.bfloat16)
    r1 = hm - hm_hi.astype(jnp.float32)
    hm_md = r1.astype(jnp.bfloat16)
    hm_lo = (r1 - hm_md.astype(jnp.float32)).astype(jnp.bfloat16)
    s_bf = s_ref[...]
    ssum = (jnp.dot(hm_hi, s_bf, preferred_element_type=jnp.float32)
            + jnp.dot(hm_md, s_bf, preferred_element_type=jnp.float32)
            + jnp.dot(hm_lo, s_bf, preferred_element_type=jnp.float32))
    sbits = jax.lax.bitcast_convert_type(ssum, jnp.int32)
    T2 = _kth_thresh(sbits, K_STRIPES, BLK)
    smask = (sbits >= T2).astype(jnp.bfloat16)
    sexp = jnp.dot(smask, st_ref[...], preferred_element_type=jnp.float32)
    hf = hm * sexp

    d = jnp.maximum(
        jnp.dot(hf, w3_ref[...], preferred_element_type=jnp.float32) + b3_ref[...], 0.0)
    out_ref[...] = jnp.maximum(
        jnp.dot(d, w4_ref[...], preferred_element_type=jnp.float32) + b4_ref[...], 0.0)


def kernel(x, W1, b1, W2, b2, W3, b3, W4, b4, boosted_scores):
    B = x.shape[0]
    grid = B // BLK
    boosts = jnp.exp(BETA * (GAMMA - boosted_scores)).reshape(1, HID)
    stripe_of = jnp.arange(HID, dtype=jnp.int32) // SD
    S = (stripe_of[:, None] == jnp.arange(NS, dtype=jnp.int32)[None, :]).astype(jnp.float32)

    full = lambda shape: pl.BlockSpec(shape, lambda i: (0, 0))
    out = pl.pallas_call(
        _fused,
        grid=(grid,),
        in_specs=[
            pl.BlockSpec((BLK, IN_DIM), lambda i: (i, 0)),
            full((IN_DIM, INTER)),
            full((1, INTER)),
            full((INTER, HID)),
            full((1, HID)),
            full((HID, INTER)),
            full((1, INTER)),
            full((INTER, IN_DIM)),
            full((1, IN_DIM)),
            full((1, HID)),
            full((HID, NS)),
            full((NS, HID)),
        ],
        out_specs=pl.BlockSpec((BLK, IN_DIM), lambda i: (i, 0)),
        out_shape=jax.ShapeDtypeStruct((B, IN_DIM), jnp.float32),
        compiler_params=pltpu.CompilerParams(
            dimension_semantics=("arbitrary",),
        ),
    )(x, W1.T, b1.reshape(1, INTER), W2.T, b2.reshape(1, HID),
      W3.T, b3.reshape(1, INTER), W4.T, b4.reshape(1, IN_DIM),
      boosts, S.astype(jnp.bfloat16), S.T.astype(jnp.bfloat16))
    return out


# speedup vs baseline: 24.9385x; 1.1493x over previous
import jax
import jax.numpy as jnp
from jax.experimental import pallas as pl
from jax.experimental.pallas import tpu as pltpu

IN_DIM = 784
INTER = 512
SD = 16
NS = 256
HID = SD * NS
K_NEURONS = 256
K_STRIPES = 32
BETA = 1.5
GAMMA = 0

BLK = 256


def _kth_thresh(bits, k, rows):
    T = jnp.zeros((rows, 1), jnp.int32)
    for b in range(30, -1, -1):
        cand = T | jnp.int32(1 << b)
        cnt = jnp.sum((bits >= cand).astype(jnp.float32), axis=1, keepdims=True)
        T = jnp.where(cnt >= k, cand, T)
    return T


def _kth_thresh16(bits, k, rows):
    one = jnp.int16(1)
    zero = jnp.int16(0)
    k16 = ((bits >> 15) - 32768).astype(jnp.int16)
    lo15 = (bits & 0x7FFF).astype(jnp.int16)

    def cnt16(mask_vals):
        m = mask_vals
        w = m.shape[1]
        while w > 128:
            w //= 2
            m = m[:, :w] + m[:, w:2 * w]
        return jnp.sum(m.astype(jnp.int32), axis=1, keepdims=True)

    U = jnp.zeros((rows, 1), jnp.int32)
    for b in range(15, -1, -1):
        cand = U | jnp.int32(1 << b)
        cand16 = (cand - 32768).astype(jnp.int16)
        cnt = cnt16(jnp.where(k16 >= cand16, one, zero))
        U = jnp.where(cnt >= k, cand, U)

    U16 = (U - 32768).astype(jnp.int16)
    eq = k16 == U16
    n_eq = cnt16(jnp.where(eq, one, zero))
    cnt_geU = cnt16(jnp.where(k16 >= U16, one, zero))
    k2 = k - (cnt_geU - n_eq)

    lo15m = jnp.where(eq, lo15, jnp.int16(-1))
    V = jnp.zeros((rows, 1), jnp.int32)
    for b in range(14, -1, -1):
        cand = V | jnp.int32(1 << b)
        cand16 = cand.astype(jnp.int16)
        cnt = cnt16(jnp.where(lo15m >= cand16, one, zero))
        V = jnp.where(cnt >= k2, cand, V)
    return (U << 15) | V


def _fused(x_ref, w1_ref, b1_ref, w2_ref, b2_ref, w3_ref, b3_ref, w4_ref,
           b4_ref, boosts_ref, s_ref, st_ref, out_ref):
    x = x_ref[...]
    h1 = jnp.maximum(
        jnp.dot(x, w1_ref[...], preferred_element_type=jnp.float32) + b1_ref[...], 0.0)
    h2 = jnp.maximum(
        jnp.dot(h1, w2_ref[...], preferred_element_type=jnp.float32) + b2_ref[...], 0.0)

    boosted = h2 * boosts_ref[...]
    bits = jax.lax.bitcast_convert_type(boosted, jnp.int32)
    T = _kth_thresh16(bits, K_NEURONS, BLK)
    hm = jnp.where(bits >= T, h2, 0.0)

    hm_hi = hm.astype(jnp.bfloat16)
    r1 = hm - hm_hi.astype(jnp.float32)
    hm_md = r1.astype(jnp.bfloat16)
    hm_lo = (r1 - hm_md.astype(jnp.float32)).astype(jnp.bfloat16)
    s_bf = s_ref[...]
    ssum = (jnp.dot(hm_hi, s_bf, preferred_element_type=jnp.float32)
            + jnp.dot(hm_md, s_bf, preferred_element_type=jnp.float32)
            + jnp.dot(hm_lo, s_bf, preferred_element_type=jnp.float32))
    sbits = jax.lax.bitcast_convert_type(ssum, jnp.int32)
    T2 = _kth_thresh(sbits, K_STRIPES, BLK)
    smask = (sbits >= T2).astype(jnp.bfloat16)
    sexp = jnp.dot(smask, st_ref[...], preferred_element_type=jnp.float32)
    hf = hm * sexp

    d = jnp.maximum(
        jnp.dot(hf, w3_ref[...], preferred_element_type=jnp.float32) + b3_ref[...], 0.0)
    out_ref[...] = jnp.maximum(
        jnp.dot(d, w4_ref[...], preferred_element_type=jnp.float32) + b4_ref[...], 0.0)


def kernel(x, W1, b1, W2, b2, W3, b3, W4, b4, boosted_scores):
    B = x.shape[0]
    grid = B // BLK
    boosts = jnp.exp(BETA * (GAMMA - boosted_scores)).reshape(1, HID)
    stripe_of = jnp.arange(HID, dtype=jnp.int32) // SD
    S = (stripe_of[:, None] == jnp.arange(NS, dtype=jnp.int32)[None, :]).astype(jnp.float32)

    full = lambda shape: pl.BlockSpec(shape, lambda i: (0, 0))
    out = pl.pallas_call(
        _fused,
        grid=(grid,),
        in_specs=[
            pl.BlockSpec((BLK, IN_DIM), lambda i: (i, 0)),
            full((IN_DIM, INTER)),
            full((1, INTER)),
            full((INTER, HID)),
            full((1, HID)),
            full((HID, INTER)),
            full((1, INTER)),
            full((INTER, IN_DIM)),
            full((1, IN_DIM)),
            full((1, HID)),
            full((HID, NS)),
            full((NS, HID)),
        ],
        out_specs=pl.BlockSpec((BLK, IN_DIM), lambda i: (i, 0)),
        out_shape=jax.ShapeDtypeStruct((B, IN_DIM), jnp.float32),
        compiler_params=pltpu.CompilerParams(
            dimension_semantics=("arbitrary",),
        ),
    )(x, W1.T, b1.reshape(1, INTER), W2.T, b2.reshape(1, HID),
      W3.T, b3.reshape(1, INTER), W4.T, b4.reshape(1, IN_DIM),
      boosts, S.astype(jnp.bfloat16), S.T.astype(jnp.bfloat16))
    return out
